# baseline (speedup 1.0000x reference)
import jax
import jax.numpy as jnp
from jax.experimental import pallas as pl
from jax.experimental.pallas import tpu as pltpu

D_MODEL = 768
N_EXPERTS = 64
TOP_K = 8
CHUNK = 1024
NBUF = 8


def _router_kernel(x_hbm, w_ref, idx_ref, wgt_ref, xbuf, sems):
    i = pl.program_id(0)
    nc = pl.num_programs(0)

    def copy(j, slot):
        return pltpu.make_async_copy(
            x_hbm.at[pl.ds(j * CHUNK, CHUNK), :], xbuf.at[slot], sems.at[slot]
        )

    @pl.when(i == 0)
    def _():
        for j in range(NBUF):
            copy(j, j).start()

    @pl.when((i > 0) & (i + NBUF - 1 < nc))
    def _():
        j = i + NBUF - 1
        copy(j, jax.lax.rem(j, NBUF)).start()

    slot = jax.lax.rem(i, NBUF)
    copy(i, slot).wait()

    x = xbuf[slot]
    w = w_ref[...]
    lt = jax.lax.dot_general(
        w, x, (((1,), (1,)), ((), ())), preferred_element_type=jnp.float32
    )
    iota = jax.lax.broadcasted_iota(jnp.int32, lt.shape, 0).astype(jnp.float32)
    cur = lt
    vals, idxs = [], []
    for _ in range(TOP_K):
        m = jnp.max(cur, axis=0, keepdims=True)
        am = jnp.min(
            jnp.where(cur == m, iota, jnp.float32(N_EXPERTS)),
            axis=0, keepdims=True,
        )
        vals.append(m)
        idxs.append(am)
        cur = jnp.where(iota == am, -jnp.inf, cur)
    v = jnp.concatenate(vals, axis=0)
    fi = jnp.concatenate(idxs, axis=0)
    e = jnp.exp(v - v[:1])
    wgt = e / jnp.sum(e, axis=0, keepdims=True)
    idx_ref[...] = fi.astype(jnp.int32)
    wgt_ref[...] = wgt


@jax.jit
def kernel(hidden_states, weight):
    b, s, h = hidden_states.shape
    n = b * s
    hs = hidden_states.reshape(n, h)
    idx, wgt = pl.pallas_call(
        _router_kernel,
        grid=(n // CHUNK,),
        in_specs=[
            pl.BlockSpec(memory_space=pltpu.MemorySpace.HBM),
            pl.BlockSpec((N_EXPERTS, h), lambda i: (0, 0)),
        ],
        out_specs=[
            pl.BlockSpec((TOP_K, CHUNK), lambda i: (0, i)),
            pl.BlockSpec((TOP_K, CHUNK), lambda i: (0, i)),
        ],
        out_shape=[
            jax.ShapeDtypeStruct((TOP_K, n), jnp.int32),
            jax.ShapeDtypeStruct((TOP_K, n), jnp.float32),
        ],
        scratch_shapes=[
            pltpu.VMEM((NBUF, CHUNK, D_MODEL), jnp.float32),
            pltpu.SemaphoreType.DMA((NBUF,)),
        ],
    )(hs, weight)
    return idx.T, wgt.T, jnp.zeros((), jnp.float32)


# speedup vs baseline: 1.0850x; 1.0850x over previous
import jax
import jax.numpy as jnp
from jax.experimental import pallas as pl
from jax.experimental.pallas import tpu as pltpu

D_MODEL = 768
N_EXPERTS = 64
TOP_K = 8
CHUNK = 2048
NBUF = 4


def _router_kernel(x_hbm, w_ref, idx_ref, wgt_ref, xbuf, sems):
    i = pl.program_id(0)
    nc = pl.num_programs(0)

    def copy(j, slot):
        return pltpu.make_async_copy(
            x_hbm.at[pl.ds(j * CHUNK, CHUNK), :], xbuf.at[slot], sems.at[slot]
        )

    @pl.when(i == 0)
    def _():
        for j in range(NBUF):
            copy(j, j).start()

    @pl.when((i > 0) & (i + NBUF - 1 < nc))
    def _():
        j = i + NBUF - 1
        copy(j, jax.lax.rem(j, NBUF)).start()

    slot = jax.lax.rem(i, NBUF)
    copy(i, slot).wait()

    x = xbuf[slot]
    w = w_ref[...]
    lt = jax.lax.dot_general(
        w, x, (((1,), (1,)), ((), ())), preferred_element_type=jnp.float32
    )
    iota = jax.lax.broadcasted_iota(jnp.int32, lt.shape, 0).astype(jnp.float32)
    cur = lt
    vals, idxs = [], []
    for _ in range(TOP_K):
        m = jnp.max(cur, axis=0, keepdims=True)
        am = jnp.min(
            jnp.where(cur == m, iota, jnp.float32(N_EXPERTS)),
            axis=0, keepdims=True,
        )
        vals.append(m)
        idxs.append(am)
        cur = jnp.where(iota == am, -jnp.inf, cur)
    v = jnp.concatenate(vals, axis=0)
    fi = jnp.concatenate(idxs, axis=0)
    e = jnp.exp(v - v[:1])
    wgt = e / jnp.sum(e, axis=0, keepdims=True)
    idx_ref[...] = fi.astype(jnp.int32)
    wgt_ref[...] = wgt


@jax.jit
def kernel(hidden_states, weight):
    b, s, h = hidden_states.shape
    n = b * s
    hs = hidden_states.reshape(n, h)
    idx, wgt = pl.pallas_call(
        _router_kernel,
        grid=(n // CHUNK,),
        in_specs=[
            pl.BlockSpec(memory_space=pltpu.MemorySpace.HBM),
            pl.BlockSpec((N_EXPERTS, h), lambda i: (0, 0)),
        ],
        out_specs=[
            pl.BlockSpec((TOP_K, CHUNK), lambda i: (0, i)),
            pl.BlockSpec((TOP_K, CHUNK), lambda i: (0, i)),
        ],
        out_shape=[
            jax.ShapeDtypeStruct((TOP_K, n), jnp.int32),
            jax.ShapeDtypeStruct((TOP_K, n), jnp.float32),
        ],
        scratch_shapes=[
            pltpu.VMEM((NBUF, CHUNK, D_MODEL), jnp.float32),
            pltpu.SemaphoreType.DMA((NBUF,)),
        ],
    )(hs, weight)
    return idx.T, wgt.T, jnp.zeros((), jnp.float32)
